# baseline (speedup 1.0000x reference)
import functools

import jax
import jax.numpy as jnp
from jax import lax
from jax.experimental import pallas as pl
from jax.experimental.pallas import tpu as pltpu
from jax.experimental.pallas import tpu_sc as plsc

N, M, F, FE, H, FD, NLAYERS, B, A = 10000, 32, 128, 16, 256, 128, 3, 100, 100

NC, NS = 2, 16
NW = NC * NS

CN = 4
CE = CN * M
NODES_PER_W = 316
NP = NW * NODES_PER_W
CHUNKS = NODES_PER_W // CN
EP = NP * M

AP = 104

HV = H // 16


def _silu(x):
    return x * jax.nn.sigmoid(x)


def _embed_tc(atom_fea, WeT, be_row):
    def body(a, w, b, o):
        o[...] = _silu(jnp.dot(a[...], w[...], preferred_element_type=jnp.float32, precision=lax.Precision.HIGHEST) + b[...])
    return pl.pallas_call(
        body,
        grid=(10,),
        in_specs=[pl.BlockSpec((N // 10, F), lambda i: (i, 0)),
                  pl.BlockSpec((F, H), lambda i: (0, 0)),
                  pl.BlockSpec((1, H), lambda i: (0, 0))],
        out_specs=pl.BlockSpec((N // 10, H), lambda i: (i, 0)),
        out_shape=jax.ShapeDtypeStruct((N, H), jnp.float32),
    )(atom_fea, WeT, be_row)


_FB = 2048


def _filters_tc(nbr_flat, f1T, b1_row, f2T, b2_row):
    def body(x, w1, b1, w2, b2, o):
        t = _silu(jnp.dot(x[...], w1[...], preferred_element_type=jnp.float32, precision=lax.Precision.HIGHEST) + b1[...])
        o[...] = jnp.dot(t, w2[...], preferred_element_type=jnp.float32, precision=lax.Precision.HIGHEST) + b2[...]
    return pl.pallas_call(
        body,
        grid=(EP // _FB,),
        in_specs=[pl.BlockSpec((_FB, FE), lambda i: (i, 0)),
                  pl.BlockSpec((FE, FD), lambda i: (0, 0)),
                  pl.BlockSpec((1, FD), lambda i: (0, 0)),
                  pl.BlockSpec((FD, H), lambda i: (0, 0)),
                  pl.BlockSpec((1, H), lambda i: (0, 0))],
        out_specs=pl.BlockSpec((_FB, H), lambda i: (i, 0)),
        out_shape=jax.ShapeDtypeStruct((EP, H), jnp.float32),
    )(nbr_flat, f1T, b1_row, f2T, b2_row)


def _update_tc(msg_padded, node, u1T, b1_row, u2T, b2_row):
    def body(m, nd, w1, b1, w2, b2, o):
        t = _silu(jnp.dot(m[...], w1[...], preferred_element_type=jnp.float32, precision=lax.Precision.HIGHEST) + b1[...])
        o[...] = _silu(nd[...] + jnp.dot(t, w2[...], preferred_element_type=jnp.float32, precision=lax.Precision.HIGHEST) + b2[...])
    return pl.pallas_call(
        body,
        grid=(10,),
        in_specs=[pl.BlockSpec((N // 10, H), lambda i: (i, 0)),
                  pl.BlockSpec((N // 10, H), lambda i: (i, 0)),
                  pl.BlockSpec((H, H), lambda i: (0, 0)),
                  pl.BlockSpec((1, H), lambda i: (0, 0)),
                  pl.BlockSpec((H, H), lambda i: (0, 0)),
                  pl.BlockSpec((1, H), lambda i: (0, 0))],
        out_specs=pl.BlockSpec((N // 10, H), lambda i: (i, 0)),
        out_shape=jax.ShapeDtypeStruct((N, H), jnp.float32),
    )(msg_padded, node, u1T, b1_row, u2T, b2_row)


def _head_tc(cmean, h1T, b1_row, h2T_pad, b2_row_pad):
    def body(x, w1, b1, w2, b2, o):
        t = _silu(jnp.dot(x[...], w1[...], preferred_element_type=jnp.float32, precision=lax.Precision.HIGHEST) + b1[...])
        o[...] = jnp.dot(t, w2[...], preferred_element_type=jnp.float32, precision=lax.Precision.HIGHEST) + b2[...]
    return pl.pallas_call(
        body,
        grid=(1,),
        in_specs=[pl.BlockSpec((B, H), lambda i: (0, 0)),
                  pl.BlockSpec((H, H), lambda i: (0, 0)),
                  pl.BlockSpec((1, H), lambda i: (0, 0)),
                  pl.BlockSpec((H, 128), lambda i: (0, 0)),
                  pl.BlockSpec((1, 128), lambda i: (0, 0))],
        out_specs=pl.BlockSpec((B, 128), lambda i: (0, 0)),
        out_shape=jax.ShapeDtypeStruct((B, 128), jnp.float32),
    )(cmean, h1T, b1_row, h2T_pad, b2_row_pad)


_MESH = plsc.VectorSubcoreMesh(core_axis_name="c", subcore_axis_name="s")


@functools.partial(
    pl.kernel,
    out_type=jax.ShapeDtypeStruct((NP, H), jnp.float32),
    mesh=_MESH,
    scratch_types=[
        pltpu.VMEM((CE,), jnp.int32),
        pltpu.VMEM((CE, H), jnp.float32),
        pltpu.VMEM((CE, H), jnp.float32),
        pltpu.VMEM((CN, H), jnp.float32),
        pltpu.SemaphoreType.DMA,
        pltpu.SemaphoreType.DMA,
    ],
)
def _message_sc(node_hbm, filt_hbm, idx_hbm, out_hbm,
                idx_v, rows_v, filt_v, msg_v, sem_g, sem_f):
    w = lax.axis_index("s") * NC + lax.axis_index("c")
    node_base = w * NODES_PER_W

    def chunk_body(t, carry):
        nb = node_base + t * CN
        eb = nb * M
        pltpu.sync_copy(idx_hbm.at[pl.ds(eb, CE)], idx_v)
        g = pltpu.async_copy(node_hbm.at[idx_v], rows_v, sem_g)
        f = pltpu.async_copy(filt_hbm.at[pl.ds(eb, CE), :], filt_v, sem_f)
        g.wait()
        f.wait()
        for n in range(CN):
            def m_body(m, acc):
                e = n * M + m
                return tuple(
                    acc[v] + rows_v[e, pl.ds(v * 16, 16)] * filt_v[e, pl.ds(v * 16, 16)]
                    for v in range(HV))
            acc0 = tuple(jnp.zeros((16,), jnp.float32) for _ in range(HV))
            acc = lax.fori_loop(0, M, m_body, acc0)
            for v in range(HV):
                msg_v[n, pl.ds(v * 16, 16)] = acc[v]
        pltpu.sync_copy(msg_v, out_hbm.at[pl.ds(nb, CN), :])
        return carry

    lax.fori_loop(0, CHUNKS, chunk_body, 0)


@functools.partial(
    pl.kernel,
    out_type=jax.ShapeDtypeStruct((B, H), jnp.float32),
    mesh=_MESH,
    scratch_types=[
        pltpu.VMEM((AP,), jnp.int32),
        pltpu.VMEM((AP, H), jnp.float32),
        pltpu.VMEM((1, H), jnp.float32),
        pltpu.SemaphoreType.DMA,
    ],
)
def _pool_sc(node_hbm, cidx_hbm, out_hbm, idx_v, rows_v, acc_v, sem):
    w = lax.axis_index("s") * NC + lax.axis_index("c")

    def crystal_body(t, carry):
        c = w + NW * t

        @pl.when(c < B)
        def _():
            pltpu.sync_copy(cidx_hbm.at[pl.ds(c * AP, AP)], idx_v)
            pltpu.async_copy(node_hbm.at[idx_v], rows_v, sem).wait()
            for v in range(HV):
                def a_body(m, acc):
                    return acc + rows_v[m, pl.ds(v * 16, 16)]
                acc = lax.fori_loop(0, A, a_body, jnp.zeros((16,), jnp.float32))
                acc_v[0, pl.ds(v * 16, 16)] = acc * (1.0 / A)
            pltpu.sync_copy(acc_v, out_hbm.at[pl.ds(c, 1), :])

        return carry

    lax.fori_loop(0, (B + NW - 1) // NW, crystal_body, 0)


def kernel(atom_fea, nbr_fea, nbr_fea_idx, crystal_atom_idx, We, be,
           fw1, fb1, fw2, fb2, uw1, ub1, uw2, ub2, hw1, hb1, hw2, hb2):
    E = N * M
    idx_pad = jnp.pad(nbr_fea_idx.astype(jnp.int32).reshape(E), (0, EP - E))
    nbr_flat = jnp.pad(nbr_fea.reshape(E, FE), ((0, EP - E), (0, 0)))
    cidx_pad = jnp.pad(crystal_atom_idx.astype(jnp.int32),
                       ((0, 0), (0, AP - A))).reshape(-1)

    node = _embed_tc(atom_fea, We.T, be.reshape(1, H))

    for i in range(NLAYERS):
        filt = _filters_tc(nbr_flat, fw1[i].T, fb1[i].reshape(1, FD),
                           fw2[i].T, fb2[i].reshape(1, H))
        msg = _message_sc(node, filt, idx_pad)
        node = _update_tc(msg, node, uw1[i].T, ub1[i].reshape(1, H),
                          uw2[i].T, ub2[i].reshape(1, H))

    cmean = _pool_sc(node, cidx_pad)
    h2T_pad = jnp.pad(hw2.T, ((0, 0), (0, 127)))
    hb2_pad = jnp.pad(hb2.reshape(1, 1), ((0, 0), (0, 127)))
    out = _head_tc(cmean, hw1.T, hb1.reshape(1, H), h2T_pad, hb2_pad)
    return out[:, 0]


# speedup vs baseline: 1.1058x; 1.1058x over previous
import functools

import jax
import jax.numpy as jnp
from jax import lax
from jax.experimental import pallas as pl
from jax.experimental.pallas import tpu as pltpu
from jax.experimental.pallas import tpu_sc as plsc

N, M, F, FE, H, FD, NLAYERS, B, A = 10000, 32, 128, 16, 256, 128, 3, 100, 100

NC, NS = 2, 16
NW = NC * NS

CN = 2
CE = CN * M
NODES_PER_W = 316
NP = NW * NODES_PER_W
CHUNKS = NODES_PER_W // CN
EP = NP * M

AP = 104

HV = H // 16


def _silu(x):
    return x * jax.nn.sigmoid(x)


def _embed_tc(atom_fea, WeT, be_row):
    def body(a, w, b, o):
        o[...] = _silu(jnp.dot(a[...], w[...], preferred_element_type=jnp.float32, precision=lax.Precision.HIGHEST) + b[...])
    return pl.pallas_call(
        body,
        grid=(10,),
        in_specs=[pl.BlockSpec((N // 10, F), lambda i: (i, 0)),
                  pl.BlockSpec((F, H), lambda i: (0, 0)),
                  pl.BlockSpec((1, H), lambda i: (0, 0))],
        out_specs=pl.BlockSpec((N // 10, H), lambda i: (i, 0)),
        out_shape=jax.ShapeDtypeStruct((N, H), jnp.float32),
    )(atom_fea, WeT, be_row)


_FB = 2048


def _filters_tc(nbr_flat, f1T, b1_row, f2T, b2_row):
    def body(x, w1, b1, w2, b2, o):
        t = _silu(jnp.dot(x[...], w1[...], preferred_element_type=jnp.float32, precision=lax.Precision.HIGHEST) + b1[...])
        o[...] = jnp.dot(t, w2[...], preferred_element_type=jnp.float32, precision=lax.Precision.HIGHEST) + b2[...]
    return pl.pallas_call(
        body,
        grid=(EP // _FB,),
        in_specs=[pl.BlockSpec((_FB, FE), lambda i: (i, 0)),
                  pl.BlockSpec((FE, FD), lambda i: (0, 0)),
                  pl.BlockSpec((1, FD), lambda i: (0, 0)),
                  pl.BlockSpec((FD, H), lambda i: (0, 0)),
                  pl.BlockSpec((1, H), lambda i: (0, 0))],
        out_specs=pl.BlockSpec((_FB, H), lambda i: (i, 0)),
        out_shape=jax.ShapeDtypeStruct((EP, H), jnp.float32),
    )(nbr_flat, f1T, b1_row, f2T, b2_row)


def _update_tc(msg_padded, node, u1T, b1_row, u2T, b2_row):
    def body(m, nd, w1, b1, w2, b2, o):
        t = _silu(jnp.dot(m[...], w1[...], preferred_element_type=jnp.float32, precision=lax.Precision.HIGHEST) + b1[...])
        o[...] = _silu(nd[...] + jnp.dot(t, w2[...], preferred_element_type=jnp.float32, precision=lax.Precision.HIGHEST) + b2[...])
    return pl.pallas_call(
        body,
        grid=(10,),
        in_specs=[pl.BlockSpec((N // 10, H), lambda i: (i, 0)),
                  pl.BlockSpec((N // 10, H), lambda i: (i, 0)),
                  pl.BlockSpec((H, H), lambda i: (0, 0)),
                  pl.BlockSpec((1, H), lambda i: (0, 0)),
                  pl.BlockSpec((H, H), lambda i: (0, 0)),
                  pl.BlockSpec((1, H), lambda i: (0, 0))],
        out_specs=pl.BlockSpec((N // 10, H), lambda i: (i, 0)),
        out_shape=jax.ShapeDtypeStruct((N, H), jnp.float32),
    )(msg_padded, node, u1T, b1_row, u2T, b2_row)


def _head_tc(cmean, h1T, b1_row, h2T_pad, b2_row_pad):
    def body(x, w1, b1, w2, b2, o):
        t = _silu(jnp.dot(x[...], w1[...], preferred_element_type=jnp.float32, precision=lax.Precision.HIGHEST) + b1[...])
        o[...] = jnp.dot(t, w2[...], preferred_element_type=jnp.float32, precision=lax.Precision.HIGHEST) + b2[...]
    return pl.pallas_call(
        body,
        grid=(1,),
        in_specs=[pl.BlockSpec((B, H), lambda i: (0, 0)),
                  pl.BlockSpec((H, H), lambda i: (0, 0)),
                  pl.BlockSpec((1, H), lambda i: (0, 0)),
                  pl.BlockSpec((H, 128), lambda i: (0, 0)),
                  pl.BlockSpec((1, 128), lambda i: (0, 0))],
        out_specs=pl.BlockSpec((B, 128), lambda i: (0, 0)),
        out_shape=jax.ShapeDtypeStruct((B, 128), jnp.float32),
    )(cmean, h1T, b1_row, h2T_pad, b2_row_pad)


_MESH = plsc.VectorSubcoreMesh(core_axis_name="c", subcore_axis_name="s")


@functools.partial(
    pl.kernel,
    out_type=jax.ShapeDtypeStruct((NP, H), jnp.float32),
    mesh=_MESH,
    scratch_types=[
        pltpu.VMEM((NODES_PER_W * M,), jnp.int32),
        pltpu.VMEM((2, CE, H), jnp.float32),
        pltpu.VMEM((2, CE, H), jnp.float32),
        pltpu.VMEM((2, CN, H), jnp.float32),
        pltpu.SemaphoreType.DMA,
        pltpu.SemaphoreType.DMA,
        pltpu.SemaphoreType.DMA,
        pltpu.SemaphoreType.DMA,
        pltpu.SemaphoreType.DMA,
        pltpu.SemaphoreType.DMA,
    ],
)
def _message_sc(node_hbm, filt_hbm, idx_hbm, out_hbm,
                idx_v, rows_v, filt_v, msg_v, sg0, sg1, sf0, sf1, ss0, ss1):
    w = lax.axis_index("s") * NC + lax.axis_index("c")
    node_base = w * NODES_PER_W
    edge_base = node_base * M
    pltpu.sync_copy(idx_hbm.at[pl.ds(edge_base, NODES_PER_W * M)], idx_v)
    sg, sf, ss = (sg0, sg1), (sf0, sf1), (ss0, ss1)

    def copies(c, b):
        el = c * CE
        g = pltpu.make_async_copy(node_hbm.at[idx_v.at[pl.ds(el, CE)]],
                                  rows_v.at[b], sg[b])
        f = pltpu.make_async_copy(filt_hbm.at[pl.ds(edge_base + el, CE), :],
                                  filt_v.at[b], sf[b])
        return g, f

    def store(c, b):
        nb = node_base + c * CN
        return pltpu.make_async_copy(msg_v.at[b], out_hbm.at[pl.ds(nb, CN), :], ss[b])

    for b in range(2):
        g, f = copies(b, b)
        g.start()
        f.start()

    def outer(t, carry):
        for b in range(2):
            c = t * 2 + b
            g, f = copies(c, b)
            g.wait()
            f.wait()

            @pl.when(t > 0)
            def _():
                store(c, b).wait()

            for n in range(CN):
                def m_body(m, acc):
                    e = n * M + m
                    return tuple(
                        acc[v] + rows_v[b, e, pl.ds(v * 16, 16)] * filt_v[b, e, pl.ds(v * 16, 16)]
                        for v in range(HV))
                acc0 = tuple(jnp.zeros((16,), jnp.float32) for _ in range(HV))
                acc = lax.fori_loop(0, M, m_body, acc0)
                for v in range(HV):
                    msg_v[b, n, pl.ds(v * 16, 16)] = acc[v]
            store(c, b).start()

            @pl.when(c + 2 < CHUNKS)
            def _():
                g2, f2 = copies(c + 2, b)
                g2.start()
                f2.start()
        return carry

    lax.fori_loop(0, CHUNKS // 2, outer, 0)
    for b in range(2):
        store(0, b).wait()


@functools.partial(
    pl.kernel,
    out_type=jax.ShapeDtypeStruct((B, H), jnp.float32),
    mesh=_MESH,
    scratch_types=[
        pltpu.VMEM((AP,), jnp.int32),
        pltpu.VMEM((AP, H), jnp.float32),
        pltpu.VMEM((1, H), jnp.float32),
        pltpu.SemaphoreType.DMA,
    ],
)
def _pool_sc(node_hbm, cidx_hbm, out_hbm, idx_v, rows_v, acc_v, sem):
    w = lax.axis_index("s") * NC + lax.axis_index("c")

    def crystal_body(t, carry):
        c = w + NW * t

        @pl.when(c < B)
        def _():
            pltpu.sync_copy(cidx_hbm.at[pl.ds(c * AP, AP)], idx_v)
            pltpu.async_copy(node_hbm.at[idx_v], rows_v, sem).wait()
            for v in range(HV):
                def a_body(m, acc):
                    return acc + rows_v[m, pl.ds(v * 16, 16)]
                acc = lax.fori_loop(0, A, a_body, jnp.zeros((16,), jnp.float32))
                acc_v[0, pl.ds(v * 16, 16)] = acc * (1.0 / A)
            pltpu.sync_copy(acc_v, out_hbm.at[pl.ds(c, 1), :])

        return carry

    lax.fori_loop(0, (B + NW - 1) // NW, crystal_body, 0)


def kernel(atom_fea, nbr_fea, nbr_fea_idx, crystal_atom_idx, We, be,
           fw1, fb1, fw2, fb2, uw1, ub1, uw2, ub2, hw1, hb1, hw2, hb2):
    E = N * M
    idx_pad = jnp.pad(nbr_fea_idx.astype(jnp.int32).reshape(E), (0, EP - E))
    nbr_flat = jnp.pad(nbr_fea.reshape(E, FE), ((0, EP - E), (0, 0)))
    cidx_pad = jnp.pad(crystal_atom_idx.astype(jnp.int32),
                       ((0, 0), (0, AP - A))).reshape(-1)

    node = _embed_tc(atom_fea, We.T, be.reshape(1, H))

    for i in range(NLAYERS):
        filt = _filters_tc(nbr_flat, fw1[i].T, fb1[i].reshape(1, FD),
                           fw2[i].T, fb2[i].reshape(1, H))
        msg = _message_sc(node, filt, idx_pad)
        node = _update_tc(msg, node, uw1[i].T, ub1[i].reshape(1, H),
                          uw2[i].T, ub2[i].reshape(1, H))

    cmean = _pool_sc(node, cidx_pad)
    h2T_pad = jnp.pad(hw2.T, ((0, 0), (0, 127)))
    hb2_pad = jnp.pad(hb2.reshape(1, 1), ((0, 0), (0, 127)))
    out = _head_tc(cmean, hw1.T, hb1.reshape(1, H), h2T_pad, hb2_pad)
    return out[:, 0]


# speedup vs baseline: 1.4484x; 1.3098x over previous
import functools

import jax
import jax.numpy as jnp
from jax import lax
from jax.experimental import pallas as pl
from jax.experimental.pallas import tpu as pltpu
from jax.experimental.pallas import tpu_sc as plsc

N, M, F, FE, H, FD, NLAYERS, B, A = 10000, 32, 128, 16, 256, 128, 3, 100, 100

NC, NS = 2, 16
NW = NC * NS

CN = 2
CE = CN * M
NODES_PER_W = 316
NP = NW * NODES_PER_W
CHUNKS = NODES_PER_W // CN
EP = NP * M

AP = 104

HV = H // 16


def _silu(x):
    return x * jax.nn.sigmoid(x)


def _dotf32(x, w):
    xh = x.astype(jnp.bfloat16)
    x1 = x - xh.astype(jnp.float32)
    xl = x1.astype(jnp.bfloat16)
    rx = (x1 - xl.astype(jnp.float32)).astype(jnp.bfloat16)
    wh = w.astype(jnp.bfloat16)
    w1 = w - wh.astype(jnp.float32)
    wl = w1.astype(jnp.bfloat16)
    rw = (w1 - wl.astype(jnp.float32)).astype(jnp.bfloat16)
    d = lambda a, b: jnp.dot(a, b, preferred_element_type=jnp.float32)
    xa = jnp.concatenate([xh, xl], axis=1)
    acc = d(xa, jnp.concatenate([wh, wl], axis=0))
    acc += d(xa, jnp.concatenate([wl, wh], axis=0))
    acc += d(jnp.concatenate([rx, xh], axis=1), jnp.concatenate([wh, rw], axis=0))
    return acc


def _embed_tc(atom_fea, WeT, be_row):
    def body(a, w, b, o):
        o[...] = _silu(jnp.dot(a[...], w[...], preferred_element_type=jnp.float32, precision=lax.Precision.HIGHEST) + b[...])
    return pl.pallas_call(
        body,
        grid=(10,),
        in_specs=[pl.BlockSpec((N // 10, F), lambda i: (i, 0)),
                  pl.BlockSpec((F, H), lambda i: (0, 0)),
                  pl.BlockSpec((1, H), lambda i: (0, 0))],
        out_specs=pl.BlockSpec((N // 10, H), lambda i: (i, 0)),
        out_shape=jax.ShapeDtypeStruct((N, H), jnp.float32),
    )(atom_fea, WeT, be_row)


_FB = 2000


def _filters_tc(nbr_flat, f1T, b1_row, f2T, b2_row):
    def body(x, w1, b1, w2, b2, o):
        t = _silu(_dotf32(x[...], w1[...]) + b1[...])
        o[...] = _dotf32(t, w2[...]) + b2[...]
    return pl.pallas_call(
        body,
        grid=(N * M // _FB,),
        in_specs=[pl.BlockSpec((_FB, FE), lambda i: (i, 0)),
                  pl.BlockSpec((FE, FD), lambda i: (0, 0)),
                  pl.BlockSpec((1, FD), lambda i: (0, 0)),
                  pl.BlockSpec((FD, H), lambda i: (0, 0)),
                  pl.BlockSpec((1, H), lambda i: (0, 0))],
        out_specs=pl.BlockSpec((_FB, H), lambda i: (i, 0)),
        out_shape=jax.ShapeDtypeStruct((EP, H), jnp.float32),
    )(nbr_flat, f1T, b1_row, f2T, b2_row)


def _update_tc(msg_padded, node, u1T, b1_row, u2T, b2_row):
    def body(m, nd, w1, b1, w2, b2, o):
        t = _silu(jnp.dot(m[...], w1[...], preferred_element_type=jnp.float32, precision=lax.Precision.HIGHEST) + b1[...])
        o[...] = _silu(nd[...] + jnp.dot(t, w2[...], preferred_element_type=jnp.float32, precision=lax.Precision.HIGHEST) + b2[...])
    return pl.pallas_call(
        body,
        grid=(10,),
        in_specs=[pl.BlockSpec((N // 10, H), lambda i: (i, 0)),
                  pl.BlockSpec((N // 10, H), lambda i: (i, 0)),
                  pl.BlockSpec((H, H), lambda i: (0, 0)),
                  pl.BlockSpec((1, H), lambda i: (0, 0)),
                  pl.BlockSpec((H, H), lambda i: (0, 0)),
                  pl.BlockSpec((1, H), lambda i: (0, 0))],
        out_specs=pl.BlockSpec((N // 10, H), lambda i: (i, 0)),
        out_shape=jax.ShapeDtypeStruct((N, H), jnp.float32),
    )(msg_padded, node, u1T, b1_row, u2T, b2_row)


def _head_tc(cmean, h1T, b1_row, h2T_pad, b2_row_pad):
    def body(x, w1, b1, w2, b2, o):
        t = _silu(jnp.dot(x[...], w1[...], preferred_element_type=jnp.float32, precision=lax.Precision.HIGHEST) + b1[...])
        o[...] = jnp.dot(t, w2[...], preferred_element_type=jnp.float32, precision=lax.Precision.HIGHEST) + b2[...]
    return pl.pallas_call(
        body,
        grid=(1,),
        in_specs=[pl.BlockSpec((B, H), lambda i: (0, 0)),
                  pl.BlockSpec((H, H), lambda i: (0, 0)),
                  pl.BlockSpec((1, H), lambda i: (0, 0)),
                  pl.BlockSpec((H, 128), lambda i: (0, 0)),
                  pl.BlockSpec((1, 128), lambda i: (0, 0))],
        out_specs=pl.BlockSpec((B, 128), lambda i: (0, 0)),
        out_shape=jax.ShapeDtypeStruct((B, 128), jnp.float32),
    )(cmean, h1T, b1_row, h2T_pad, b2_row_pad)


_MESH = plsc.VectorSubcoreMesh(core_axis_name="c", subcore_axis_name="s")


@functools.partial(
    pl.kernel,
    out_type=jax.ShapeDtypeStruct((NP, H), jnp.float32),
    mesh=_MESH,
    scratch_types=[
        pltpu.VMEM((NODES_PER_W * M,), jnp.int32),
        pltpu.VMEM((2, CE, H), jnp.float32),
        pltpu.VMEM((2, CE, H), jnp.float32),
        pltpu.VMEM((2, CN, H), jnp.float32),
        pltpu.SemaphoreType.DMA,
        pltpu.SemaphoreType.DMA,
        pltpu.SemaphoreType.DMA,
        pltpu.SemaphoreType.DMA,
        pltpu.SemaphoreType.DMA,
        pltpu.SemaphoreType.DMA,
    ],
)
def _message_sc(node_hbm, filt_hbm, idx_hbm, out_hbm,
                idx_v, rows_v, filt_v, msg_v, sg0, sg1, sf0, sf1, ss0, ss1):
    w = lax.axis_index("s") * NC + lax.axis_index("c")
    node_base = w * NODES_PER_W
    edge_base = node_base * M
    pltpu.sync_copy(idx_hbm.at[pl.ds(edge_base, NODES_PER_W * M)], idx_v)
    sg, sf, ss = (sg0, sg1), (sf0, sf1), (ss0, ss1)

    def copies(c, b):
        el = c * CE
        g = pltpu.make_async_copy(node_hbm.at[idx_v.at[pl.ds(el, CE)]],
                                  rows_v.at[b], sg[b])
        f = pltpu.make_async_copy(filt_hbm.at[pl.ds(edge_base + el, CE), :],
                                  filt_v.at[b], sf[b])
        return g, f

    def store(c, b):
        nb = node_base + c * CN
        return pltpu.make_async_copy(msg_v.at[b], out_hbm.at[pl.ds(nb, CN), :], ss[b])

    for b in range(2):
        g, f = copies(b, b)
        g.start()
        f.start()

    def outer(t, carry):
        for b in range(2):
            c = t * 2 + b
            g, f = copies(c, b)
            g.wait()
            f.wait()

            @pl.when(t > 0)
            def _():
                store(c, b).wait()

            for n in range(CN):
                def m_body(m, acc):
                    e = n * M + m
                    return tuple(
                        acc[v] + rows_v[b, e, pl.ds(v * 16, 16)] * filt_v[b, e, pl.ds(v * 16, 16)]
                        for v in range(HV))
                acc0 = tuple(jnp.zeros((16,), jnp.float32) for _ in range(HV))
                acc = lax.fori_loop(0, M, m_body, acc0)
                for v in range(HV):
                    msg_v[b, n, pl.ds(v * 16, 16)] = acc[v]
            store(c, b).start()

            @pl.when(c + 2 < CHUNKS)
            def _():
                g2, f2 = copies(c + 2, b)
                g2.start()
                f2.start()
        return carry

    lax.fori_loop(0, CHUNKS // 2, outer, 0)
    for b in range(2):
        store(0, b).wait()


@functools.partial(
    pl.kernel,
    out_type=jax.ShapeDtypeStruct((B, H), jnp.float32),
    mesh=_MESH,
    scratch_types=[
        pltpu.VMEM((AP,), jnp.int32),
        pltpu.VMEM((AP, H), jnp.float32),
        pltpu.VMEM((1, H), jnp.float32),
        pltpu.SemaphoreType.DMA,
    ],
)
def _pool_sc(node_hbm, cidx_hbm, out_hbm, idx_v, rows_v, acc_v, sem):
    w = lax.axis_index("s") * NC + lax.axis_index("c")

    def crystal_body(t, carry):
        c = w + NW * t

        @pl.when(c < B)
        def _():
            pltpu.sync_copy(cidx_hbm.at[pl.ds(c * AP, AP)], idx_v)
            pltpu.async_copy(node_hbm.at[idx_v], rows_v, sem).wait()
            for v in range(HV):
                def a_body(m, acc):
                    return acc + rows_v[m, pl.ds(v * 16, 16)]
                acc = lax.fori_loop(0, A, a_body, jnp.zeros((16,), jnp.float32))
                acc_v[0, pl.ds(v * 16, 16)] = acc * (1.0 / A)
            pltpu.sync_copy(acc_v, out_hbm.at[pl.ds(c, 1), :])

        return carry

    lax.fori_loop(0, (B + NW - 1) // NW, crystal_body, 0)


def kernel(atom_fea, nbr_fea, nbr_fea_idx, crystal_atom_idx, We, be,
           fw1, fb1, fw2, fb2, uw1, ub1, uw2, ub2, hw1, hb1, hw2, hb2):
    E = N * M
    idx_pad = jnp.pad(nbr_fea_idx.astype(jnp.int32).reshape(E), (0, EP - E))
    nbr_flat = nbr_fea.reshape(E, FE)
    cidx_pad = jnp.pad(crystal_atom_idx.astype(jnp.int32),
                       ((0, 0), (0, AP - A))).reshape(-1)

    node = _embed_tc(atom_fea, We.T, be.reshape(1, H))

    for i in range(NLAYERS):
        filt = _filters_tc(nbr_flat, fw1[i].T, fb1[i].reshape(1, FD),
                           fw2[i].T, fb2[i].reshape(1, H))
        msg = _message_sc(node, filt, idx_pad)
        node = _update_tc(msg, node, uw1[i].T, ub1[i].reshape(1, H),
                          uw2[i].T, ub2[i].reshape(1, H))

    cmean = _pool_sc(node, cidx_pad)
    h2T_pad = jnp.pad(hw2.T, ((0, 0), (0, 127)))
    hb2_pad = jnp.pad(hb2.reshape(1, 1), ((0, 0), (0, 127)))
    out = _head_tc(cmean, hw1.T, hb1.reshape(1, H), h2T_pad, hb2_pad)
    return out[:, 0]


# speedup vs baseline: 1.4748x; 1.0183x over previous
import functools

import jax
import jax.numpy as jnp
from jax import lax
from jax.experimental import pallas as pl
from jax.experimental.pallas import tpu as pltpu
from jax.experimental.pallas import tpu_sc as plsc

N, M, F, FE, H, FD, NLAYERS, B, A = 10000, 32, 128, 16, 256, 128, 3, 100, 100

NC, NS = 2, 16
NW = NC * NS

CN = 2
CE = CN * M
NODES_PER_W = 316
NP = NW * NODES_PER_W
CHUNKS = NODES_PER_W // CN
EP = NP * M

AP = 104

HV = H // 16


def _silu(x):
    return x * jax.nn.sigmoid(x)


def _dotf32(x, w):
    xh = x.astype(jnp.bfloat16)
    x1 = x - xh.astype(jnp.float32)
    xl = x1.astype(jnp.bfloat16)
    rx = (x1 - xl.astype(jnp.float32)).astype(jnp.bfloat16)
    wh = w.astype(jnp.bfloat16)
    w1 = w - wh.astype(jnp.float32)
    wl = w1.astype(jnp.bfloat16)
    rw = (w1 - wl.astype(jnp.float32)).astype(jnp.bfloat16)
    d = lambda a, b: jnp.dot(a, b, preferred_element_type=jnp.float32)
    xa = jnp.concatenate([xh, xl], axis=1)
    acc = d(xa, jnp.concatenate([wh, wl], axis=0))
    acc += d(xa, jnp.concatenate([wl, wh], axis=0))
    acc += d(jnp.concatenate([rx, xh], axis=1), jnp.concatenate([wh, rw], axis=0))
    return acc


def _embed_tc(atom_fea, WeT, be_row):
    def body(a, w, b, o):
        o[...] = _silu(jnp.dot(a[...], w[...], preferred_element_type=jnp.float32, precision=lax.Precision.HIGHEST) + b[...])
    return pl.pallas_call(
        body,
        grid=(10,),
        in_specs=[pl.BlockSpec((N // 10, F), lambda i: (i, 0)),
                  pl.BlockSpec((F, H), lambda i: (0, 0)),
                  pl.BlockSpec((1, H), lambda i: (0, 0))],
        out_specs=pl.BlockSpec((N // 10, H), lambda i: (i, 0)),
        out_shape=jax.ShapeDtypeStruct((N, H), jnp.float32),
    )(atom_fea, WeT, be_row)


_FB = 2000


def _filters_tc(nbr_flat, f1T, b1_row, f2T, b2_row):
    def body(x, w1, b1, w2, b2, o):
        xp = jnp.concatenate([x[...], jnp.zeros((_FB, FD - FE), jnp.float32)], axis=1)
        t = _silu(_dotf32(xp, w1[...]) + b1[...])
        o[...] = jnp.dot(t, w2[...], preferred_element_type=jnp.float32) + b2[...]
    return pl.pallas_call(
        body,
        grid=(N * M // _FB,),
        in_specs=[pl.BlockSpec((_FB, FE), lambda i: (i, 0)),
                  pl.BlockSpec((FD, FD), lambda i: (0, 0)),
                  pl.BlockSpec((1, FD), lambda i: (0, 0)),
                  pl.BlockSpec((FD, H), lambda i: (0, 0)),
                  pl.BlockSpec((1, H), lambda i: (0, 0))],
        out_specs=pl.BlockSpec((_FB, H), lambda i: (i, 0)),
        out_shape=jax.ShapeDtypeStruct((EP, H), jnp.float32),
    )(nbr_flat, f1T, b1_row, f2T, b2_row)


def _update_tc(msg_padded, node, u1T, b1_row, u2T, b2_row):
    def body(m, nd, w1, b1, w2, b2, o):
        t = _silu(jnp.dot(m[...], w1[...], preferred_element_type=jnp.float32, precision=lax.Precision.HIGHEST) + b1[...])
        o[...] = _silu(nd[...] + jnp.dot(t, w2[...], preferred_element_type=jnp.float32, precision=lax.Precision.HIGHEST) + b2[...])
    return pl.pallas_call(
        body,
        grid=(10,),
        in_specs=[pl.BlockSpec((N // 10, H), lambda i: (i, 0)),
                  pl.BlockSpec((N // 10, H), lambda i: (i, 0)),
                  pl.BlockSpec((H, H), lambda i: (0, 0)),
                  pl.BlockSpec((1, H), lambda i: (0, 0)),
                  pl.BlockSpec((H, H), lambda i: (0, 0)),
                  pl.BlockSpec((1, H), lambda i: (0, 0))],
        out_specs=pl.BlockSpec((N // 10, H), lambda i: (i, 0)),
        out_shape=jax.ShapeDtypeStruct((N, H), jnp.float32),
    )(msg_padded, node, u1T, b1_row, u2T, b2_row)


def _head_tc(cmean, h1T, b1_row, h2T_pad, b2_row_pad):
    def body(x, w1, b1, w2, b2, o):
        t = _silu(jnp.dot(x[...], w1[...], preferred_element_type=jnp.float32, precision=lax.Precision.HIGHEST) + b1[...])
        o[...] = jnp.dot(t, w2[...], preferred_element_type=jnp.float32, precision=lax.Precision.HIGHEST) + b2[...]
    return pl.pallas_call(
        body,
        grid=(1,),
        in_specs=[pl.BlockSpec((B, H), lambda i: (0, 0)),
                  pl.BlockSpec((H, H), lambda i: (0, 0)),
                  pl.BlockSpec((1, H), lambda i: (0, 0)),
                  pl.BlockSpec((H, 128), lambda i: (0, 0)),
                  pl.BlockSpec((1, 128), lambda i: (0, 0))],
        out_specs=pl.BlockSpec((B, 128), lambda i: (0, 0)),
        out_shape=jax.ShapeDtypeStruct((B, 128), jnp.float32),
    )(cmean, h1T, b1_row, h2T_pad, b2_row_pad)


_MESH = plsc.VectorSubcoreMesh(core_axis_name="c", subcore_axis_name="s")


@functools.partial(
    pl.kernel,
    out_type=jax.ShapeDtypeStruct((NP, H), jnp.float32),
    mesh=_MESH,
    scratch_types=[
        pltpu.VMEM((NODES_PER_W * M,), jnp.int32),
        pltpu.VMEM((2, CE, H), jnp.float32),
        pltpu.VMEM((2, CE, H), jnp.float32),
        pltpu.VMEM((2, CN, H), jnp.float32),
        pltpu.SemaphoreType.DMA,
        pltpu.SemaphoreType.DMA,
        pltpu.SemaphoreType.DMA,
        pltpu.SemaphoreType.DMA,
        pltpu.SemaphoreType.DMA,
        pltpu.SemaphoreType.DMA,
    ],
)
def _message_sc(node_hbm, filt_hbm, idx_hbm, out_hbm,
                idx_v, rows_v, filt_v, msg_v, sg0, sg1, sf0, sf1, ss0, ss1):
    w = lax.axis_index("s") * NC + lax.axis_index("c")
    node_base = w * NODES_PER_W
    edge_base = node_base * M
    pltpu.sync_copy(idx_hbm.at[pl.ds(edge_base, NODES_PER_W * M)], idx_v)
    sg, sf, ss = (sg0, sg1), (sf0, sf1), (ss0, ss1)

    def copies(c, b):
        el = c * CE
        g = pltpu.make_async_copy(node_hbm.at[idx_v.at[pl.ds(el, CE)]],
                                  rows_v.at[b], sg[b])
        f = pltpu.make_async_copy(filt_hbm.at[pl.ds(edge_base + el, CE), :],
                                  filt_v.at[b], sf[b])
        return g, f

    def store(c, b):
        nb = node_base + c * CN
        return pltpu.make_async_copy(msg_v.at[b], out_hbm.at[pl.ds(nb, CN), :], ss[b])

    for b in range(2):
        g, f = copies(b, b)
        g.start()
        f.start()

    def outer(t, carry):
        for b in range(2):
            c = t * 2 + b
            g, f = copies(c, b)
            g.wait()
            f.wait()

            @pl.when(t > 0)
            def _():
                store(c, b).wait()

            for n in range(CN):
                def m_body(m, acc):
                    e = n * M + m
                    return tuple(
                        acc[v] + rows_v[b, e, pl.ds(v * 16, 16)] * filt_v[b, e, pl.ds(v * 16, 16)]
                        for v in range(HV))
                acc0 = tuple(jnp.zeros((16,), jnp.float32) for _ in range(HV))
                acc = lax.fori_loop(0, M, m_body, acc0)
                for v in range(HV):
                    msg_v[b, n, pl.ds(v * 16, 16)] = acc[v]
            store(c, b).start()

            @pl.when(c + 2 < CHUNKS)
            def _():
                g2, f2 = copies(c + 2, b)
                g2.start()
                f2.start()
        return carry

    lax.fori_loop(0, CHUNKS // 2, outer, 0)
    for b in range(2):
        store(0, b).wait()


@functools.partial(
    pl.kernel,
    out_type=jax.ShapeDtypeStruct((B, H), jnp.float32),
    mesh=_MESH,
    scratch_types=[
        pltpu.VMEM((AP,), jnp.int32),
        pltpu.VMEM((AP, H), jnp.float32),
        pltpu.VMEM((1, H), jnp.float32),
        pltpu.SemaphoreType.DMA,
    ],
)
def _pool_sc(node_hbm, cidx_hbm, out_hbm, idx_v, rows_v, acc_v, sem):
    w = lax.axis_index("s") * NC + lax.axis_index("c")

    def crystal_body(t, carry):
        c = w + NW * t

        @pl.when(c < B)
        def _():
            pltpu.sync_copy(cidx_hbm.at[pl.ds(c * AP, AP)], idx_v)
            pltpu.async_copy(node_hbm.at[idx_v], rows_v, sem).wait()
            for v in range(HV):
                def a_body(m, acc):
                    return acc + rows_v[m, pl.ds(v * 16, 16)]
                acc = lax.fori_loop(0, A, a_body, jnp.zeros((16,), jnp.float32))
                acc_v[0, pl.ds(v * 16, 16)] = acc * (1.0 / A)
            pltpu.sync_copy(acc_v, out_hbm.at[pl.ds(c, 1), :])

        return carry

    lax.fori_loop(0, (B + NW - 1) // NW, crystal_body, 0)


def kernel(atom_fea, nbr_fea, nbr_fea_idx, crystal_atom_idx, We, be,
           fw1, fb1, fw2, fb2, uw1, ub1, uw2, ub2, hw1, hb1, hw2, hb2):
    E = N * M
    idx_pad = jnp.pad(nbr_fea_idx.astype(jnp.int32).reshape(E), (0, EP - E))
    nbr_flat = nbr_fea.reshape(E, FE)
    cidx_pad = jnp.pad(crystal_atom_idx.astype(jnp.int32),
                       ((0, 0), (0, AP - A))).reshape(-1)

    node = _embed_tc(atom_fea, We.T, be.reshape(1, H))

    for i in range(NLAYERS):
        f1T_pad = jnp.pad(fw1[i].T, ((0, FD - FE), (0, 0)))
        filt = _filters_tc(nbr_flat, f1T_pad, fb1[i].reshape(1, FD),
                           fw2[i].T, fb2[i].reshape(1, H))
        msg = _message_sc(node, filt, idx_pad)
        node = _update_tc(msg, node, uw1[i].T, ub1[i].reshape(1, H),
                          uw2[i].T, ub2[i].reshape(1, H))

    cmean = _pool_sc(node, cidx_pad)
    h2T_pad = jnp.pad(hw2.T, ((0, 0), (0, 127)))
    hb2_pad = jnp.pad(hb2.reshape(1, 1), ((0, 0), (0, 127)))
    out = _head_tc(cmean, hw1.T, hb1.reshape(1, H), h2T_pad, hb2_pad)
    return out[:, 0]


# speedup vs baseline: 1.5671x; 1.0626x over previous
import functools

import jax
import jax.numpy as jnp
from jax import lax
from jax.experimental import pallas as pl
from jax.experimental.pallas import tpu as pltpu
from jax.experimental.pallas import tpu_sc as plsc

N, M, F, FE, H, FD, NLAYERS, B, A = 10000, 32, 128, 16, 256, 128, 3, 100, 100

NC, NS = 2, 16
NW = NC * NS

CN = 2
CE = CN * M
NODES_PER_W = 316
NP = NW * NODES_PER_W
CHUNKS = NODES_PER_W // CN
EP = NP * M

AP = 104

HV = H // 16


def _silu(x):
    return x * jax.nn.sigmoid(x)


def _dotf32(x, w):
    xh = x.astype(jnp.bfloat16)
    x1 = x - xh.astype(jnp.float32)
    xl = x1.astype(jnp.bfloat16)
    rx = (x1 - xl.astype(jnp.float32)).astype(jnp.bfloat16)
    wh = w.astype(jnp.bfloat16)
    w1 = w - wh.astype(jnp.float32)
    wl = w1.astype(jnp.bfloat16)
    rw = (w1 - wl.astype(jnp.float32)).astype(jnp.bfloat16)
    d = lambda a, b: jnp.dot(a, b, preferred_element_type=jnp.float32)
    xa = jnp.concatenate([xh, xl], axis=1)
    acc = d(xa, jnp.concatenate([wh, wl], axis=0))
    acc += d(xa, jnp.concatenate([wl, wh], axis=0))
    acc += d(jnp.concatenate([rx, xh], axis=1), jnp.concatenate([wh, rw], axis=0))
    return acc


def _embed_tc(atom_fea, WeT, be_row):
    def body(a, w, b, o):
        o[...] = _silu(jnp.dot(a[...], w[...], preferred_element_type=jnp.float32, precision=lax.Precision.HIGHEST) + b[...])
    return pl.pallas_call(
        body,
        grid=(10,),
        in_specs=[pl.BlockSpec((N // 10, F), lambda i: (i, 0)),
                  pl.BlockSpec((F, H), lambda i: (0, 0)),
                  pl.BlockSpec((1, H), lambda i: (0, 0))],
        out_specs=pl.BlockSpec((N // 10, H), lambda i: (i, 0)),
        out_shape=jax.ShapeDtypeStruct((N, H), jnp.float32),
    )(atom_fea, WeT, be_row)


_FB = 2000


def _filters_tc(nbr_flat, f1T, b1_row, f2T, b2_row):
    def body(x, w1, b1, w2, b2, o):
        t = _silu(jnp.dot(x[...], w1[...], preferred_element_type=jnp.float32) + b1[...])
        o[...] = jnp.dot(t, w2[...], preferred_element_type=jnp.float32) + b2[...]
    return pl.pallas_call(
        body,
        grid=(N * M // _FB,),
        in_specs=[pl.BlockSpec((_FB, FE), lambda i: (i, 0)),
                  pl.BlockSpec((FE, FD), lambda i: (0, 0)),
                  pl.BlockSpec((1, FD), lambda i: (0, 0)),
                  pl.BlockSpec((FD, H), lambda i: (0, 0)),
                  pl.BlockSpec((1, H), lambda i: (0, 0))],
        out_specs=pl.BlockSpec((_FB, H), lambda i: (i, 0)),
        out_shape=jax.ShapeDtypeStruct((EP, H), jnp.float32),
    )(nbr_flat, f1T, b1_row, f2T, b2_row)


def _update_tc(msg_padded, node, u1T, b1_row, u2T, b2_row):
    def body(m, nd, w1, b1, w2, b2, o):
        t = _silu(jnp.dot(m[...], w1[...], preferred_element_type=jnp.float32) + b1[...])
        o[...] = _silu(nd[...] + jnp.dot(t, w2[...], preferred_element_type=jnp.float32) + b2[...])
    return pl.pallas_call(
        body,
        grid=(10,),
        in_specs=[pl.BlockSpec((N // 10, H), lambda i: (i, 0)),
                  pl.BlockSpec((N // 10, H), lambda i: (i, 0)),
                  pl.BlockSpec((H, H), lambda i: (0, 0)),
                  pl.BlockSpec((1, H), lambda i: (0, 0)),
                  pl.BlockSpec((H, H), lambda i: (0, 0)),
                  pl.BlockSpec((1, H), lambda i: (0, 0))],
        out_specs=pl.BlockSpec((N // 10, H), lambda i: (i, 0)),
        out_shape=jax.ShapeDtypeStruct((N, H), jnp.float32),
    )(msg_padded, node, u1T, b1_row, u2T, b2_row)


def _head_tc(cmean, h1T, b1_row, h2T_pad, b2_row_pad):
    def body(x, w1, b1, w2, b2, o):
        t = _silu(jnp.dot(x[...], w1[...], preferred_element_type=jnp.float32, precision=lax.Precision.HIGHEST) + b1[...])
        o[...] = jnp.dot(t, w2[...], preferred_element_type=jnp.float32, precision=lax.Precision.HIGHEST) + b2[...]
    return pl.pallas_call(
        body,
        grid=(1,),
        in_specs=[pl.BlockSpec((B, H), lambda i: (0, 0)),
                  pl.BlockSpec((H, H), lambda i: (0, 0)),
                  pl.BlockSpec((1, H), lambda i: (0, 0)),
                  pl.BlockSpec((H, 128), lambda i: (0, 0)),
                  pl.BlockSpec((1, 128), lambda i: (0, 0))],
        out_specs=pl.BlockSpec((B, 128), lambda i: (0, 0)),
        out_shape=jax.ShapeDtypeStruct((B, 128), jnp.float32),
    )(cmean, h1T, b1_row, h2T_pad, b2_row_pad)


_MESH = plsc.VectorSubcoreMesh(core_axis_name="c", subcore_axis_name="s")


@functools.partial(
    pl.kernel,
    out_type=jax.ShapeDtypeStruct((NP, H), jnp.float32),
    mesh=_MESH,
    scratch_types=[
        pltpu.VMEM((NODES_PER_W * M,), jnp.int32),
        pltpu.VMEM((2, CE, H), jnp.float32),
        pltpu.VMEM((2, CE, H), jnp.float32),
        pltpu.VMEM((2, CN, H), jnp.float32),
        pltpu.SemaphoreType.DMA,
        pltpu.SemaphoreType.DMA,
        pltpu.SemaphoreType.DMA,
        pltpu.SemaphoreType.DMA,
        pltpu.SemaphoreType.DMA,
        pltpu.SemaphoreType.DMA,
    ],
)
def _message_sc(node_hbm, filt_hbm, idx_hbm, out_hbm,
                idx_v, rows_v, filt_v, msg_v, sg0, sg1, sf0, sf1, ss0, ss1):
    w = lax.axis_index("s") * NC + lax.axis_index("c")
    node_base = w * NODES_PER_W
    edge_base = node_base * M
    pltpu.sync_copy(idx_hbm.at[pl.ds(edge_base, NODES_PER_W * M)], idx_v)
    sg, sf, ss = (sg0, sg1), (sf0, sf1), (ss0, ss1)

    def copies(c, b):
        el = c * CE
        g = pltpu.make_async_copy(node_hbm.at[idx_v.at[pl.ds(el, CE)]],
                                  rows_v.at[b], sg[b])
        f = pltpu.make_async_copy(filt_hbm.at[pl.ds(edge_base + el, CE), :],
                                  filt_v.at[b], sf[b])
        return g, f

    def store(c, b):
        nb = node_base + c * CN
        return pltpu.make_async_copy(msg_v.at[b], out_hbm.at[pl.ds(nb, CN), :], ss[b])

    for b in range(2):
        g, f = copies(b, b)
        g.start()
        f.start()

    def outer(t, carry):
        for b in range(2):
            c = t * 2 + b
            g, f = copies(c, b)
            g.wait()
            f.wait()

            @pl.when(t > 0)
            def _():
                store(c, b).wait()

            for n in range(CN):
                def m_body(m, acc):
                    e = n * M + m
                    return tuple(
                        acc[v] + rows_v[b, e, pl.ds(v * 16, 16)] * filt_v[b, e, pl.ds(v * 16, 16)]
                        for v in range(HV))
                acc0 = tuple(jnp.zeros((16,), jnp.float32) for _ in range(HV))
                acc = lax.fori_loop(0, M, m_body, acc0)
                for v in range(HV):
                    msg_v[b, n, pl.ds(v * 16, 16)] = acc[v]
            store(c, b).start()

            @pl.when(c + 2 < CHUNKS)
            def _():
                g2, f2 = copies(c + 2, b)
                g2.start()
                f2.start()
        return carry

    lax.fori_loop(0, CHUNKS // 2, outer, 0)
    for b in range(2):
        store(0, b).wait()


@functools.partial(
    pl.kernel,
    out_type=jax.ShapeDtypeStruct((B, H), jnp.float32),
    mesh=_MESH,
    scratch_types=[
        pltpu.VMEM((AP,), jnp.int32),
        pltpu.VMEM((AP, H), jnp.float32),
        pltpu.VMEM((1, H), jnp.float32),
        pltpu.SemaphoreType.DMA,
    ],
)
def _pool_sc(node_hbm, cidx_hbm, out_hbm, idx_v, rows_v, acc_v, sem):
    w = lax.axis_index("s") * NC + lax.axis_index("c")

    def crystal_body(t, carry):
        c = w + NW * t

        @pl.when(c < B)
        def _():
            pltpu.sync_copy(cidx_hbm.at[pl.ds(c * AP, AP)], idx_v)
            pltpu.async_copy(node_hbm.at[idx_v], rows_v, sem).wait()
            for v in range(HV):
                def a_body(m, acc):
                    return acc + rows_v[m, pl.ds(v * 16, 16)]
                acc = lax.fori_loop(0, A, a_body, jnp.zeros((16,), jnp.float32))
                acc_v[0, pl.ds(v * 16, 16)] = acc * (1.0 / A)
            pltpu.sync_copy(acc_v, out_hbm.at[pl.ds(c, 1), :])

        return carry

    lax.fori_loop(0, (B + NW - 1) // NW, crystal_body, 0)


def kernel(atom_fea, nbr_fea, nbr_fea_idx, crystal_atom_idx, We, be,
           fw1, fb1, fw2, fb2, uw1, ub1, uw2, ub2, hw1, hb1, hw2, hb2):
    E = N * M
    idx_pad = jnp.pad(nbr_fea_idx.astype(jnp.int32).reshape(E), (0, EP - E))
    nbr_flat = nbr_fea.reshape(E, FE)
    cidx_pad = jnp.pad(crystal_atom_idx.astype(jnp.int32),
                       ((0, 0), (0, AP - A))).reshape(-1)

    node = _embed_tc(atom_fea, We.T, be.reshape(1, H))

    for i in range(NLAYERS):
        filt = _filters_tc(nbr_flat, fw1[i].T, fb1[i].reshape(1, FD),
                           fw2[i].T, fb2[i].reshape(1, H))
        msg = _message_sc(node, filt, idx_pad)
        node = _update_tc(msg, node, uw1[i].T, ub1[i].reshape(1, H),
                          uw2[i].T, ub2[i].reshape(1, H))

    cmean = _pool_sc(node, cidx_pad)
    h2T_pad = jnp.pad(hw2.T, ((0, 0), (0, 127)))
    hb2_pad = jnp.pad(hb2.reshape(1, 1), ((0, 0), (0, 127)))
    out = _head_tc(cmean, hw1.T, hb1.reshape(1, H), h2T_pad, hb2_pad)
    return out[:, 0]
